# baseline (speedup 1.0000x reference)
import functools

import jax
import jax.numpy as jnp
import numpy as np
from jax.experimental import pallas as pl
from jax.experimental.pallas import tpu as pltpu

B, S, D = 1, 2048, 1024
H, KV, HD = 16, 4, 64
E, K, DFF = 8, 2, 2048
C = 512
EC = E * C
CLIP = 8.0
ROPE = 500000.0
EPS = 1e-5
REP = H // KV
HALF = HD // 2

BSQ = 256
QKVW = D + 2 * KV * HD
FB = 512
SB = 512

_f32 = jnp.float32


def _roll_lanes(t, sh):
    return jnp.concatenate([t[:, sh:], t[:, :sh]], axis=1)


def _rope(t, cs, ss, nlanes, scale=1.0):
    within = jax.lax.broadcasted_iota(jnp.int32, (1, nlanes), 1) % HD
    reps = nlanes // HALF
    cosv = jnp.concatenate([cs] * reps, axis=1)
    sinv = jnp.concatenate([ss] * reps, axis=1)
    rot = jnp.where(within < HALF, -_roll_lanes(t, HALF), _roll_lanes(t, nlanes - HALF))
    return (t * cosv + rot * sinv) * _f32(scale)


def _prologue_kern(x_ref, pos_ref, g1_ref, wqkv_ref, q_ref, kt_ref, v3_ref):
    x = x_ref[...]
    mu = jnp.mean(x, axis=-1, keepdims=True)
    var = jnp.mean((x - mu) ** 2, axis=-1, keepdims=True)
    h = (x - mu) * jax.lax.rsqrt(var + EPS) * g1_ref[...]
    qkv = jnp.dot(h, wqkv_ref[...], preferred_element_type=_f32)
    qkv = jnp.clip(qkv, -CLIP, CLIP)
    pos_f = pos_ref[...].astype(_f32)
    j = jax.lax.broadcasted_iota(jnp.int32, (1, HALF), 1).astype(_f32)
    inv = jnp.exp(j * _f32(-np.log(ROPE) / HALF))
    theta = pos_f * inv
    cs = jnp.cos(theta)
    ss = jnp.sin(theta)
    q_ref[...] = _rope(qkv[:, :D], cs, ss, D, scale=1.0 / np.sqrt(HD))
    k = _rope(qkv[:, D:D + KV * HD], cs, ss, KV * HD)
    kt_ref[...] = k.T
    v = qkv[:, D + KV * HD:]
    v3_ref[...] = jnp.concatenate(
        [v[:, kh * HD:(kh + 1) * HD][None] for kh in range(KV)], axis=0)


BQ = 512
BK = 256
NQB = S // BQ
NJ = S // BK
HP = H // 2


def _attn_kern(q_ref, kt_ref, v_ref, o_ref, acc_scr, m0_scr, m1_scr, l0_scr, l1_scr):
    qb = pl.program_id(1)
    j = pl.program_id(2)
    last_j = (qb + 1) * (BQ // BK) - 1

    @pl.when(j == 0)
    def _():
        m0_scr[...] = jnp.full((BQ, 1), _f32(-1e30))
        m1_scr[...] = jnp.full((BQ, 1), _f32(-1e30))
        l0_scr[...] = jnp.zeros((BQ, 1), _f32)
        l1_scr[...] = jnp.zeros((BQ, 1), _f32)

    @pl.when(j <= last_j)
    def _():
        row = jax.lax.broadcasted_iota(jnp.int32, (BQ, BK), 0) + qb * BQ
        col = jax.lax.broadcasted_iota(jnp.int32, (BQ, BK), 1) + j * BK
        causal = col <= row
        kt = kt_ref[0]
        vj = v_ref[0]
        for hh, (m_scr, l_scr) in enumerate(((m0_scr, l0_scr), (m1_scr, l1_scr))):
            qh = q_ref[:, hh * HD:(hh + 1) * HD]
            s = jnp.dot(qh, kt, preferred_element_type=_f32)
            s = jnp.where(causal, s, _f32(-1e9))
            m_old = m_scr[...]
            m_new = jnp.maximum(m_old, jnp.max(s, axis=-1, keepdims=True))
            corr = jnp.exp(m_old - m_new)
            p = jnp.exp(s - m_new)
            pv = jnp.dot(p, vj, preferred_element_type=_f32)
            sl = slice(hh * HD, (hh + 1) * HD)

            @pl.when(j == 0)
            def _():
                acc_scr[:, sl] = pv

            @pl.when(j > 0)
            def _():
                acc_scr[:, sl] = acc_scr[:, sl] * corr + pv

            l_scr[...] = l_scr[...] * corr + jnp.sum(p, axis=-1, keepdims=True)
            m_scr[...] = m_new

    @pl.when(j == last_j)
    def _():
        o_ref[...] = acc_scr[...] / jnp.concatenate(
            [jnp.broadcast_to(l0_scr[...], (BQ, HD)),
             jnp.broadcast_to(l1_scr[...], (BQ, HD))], axis=1)


def _proj_kern(attn_ref, wo_ref, res_ref, g2_ref, wr_ref, h_ref, x2_ref, lg_ref):
    hh = res_ref[...] + jnp.dot(attn_ref[...], wo_ref[...], preferred_element_type=_f32)
    h_ref[...] = hh
    mu = jnp.mean(hh, axis=-1, keepdims=True)
    var = jnp.mean((hh - mu) ** 2, axis=-1, keepdims=True)
    x2 = (hh - mu) * jax.lax.rsqrt(var + EPS) * g2_ref[...]
    x2_ref[...] = x2
    lg_ref[...] = jnp.dot(x2, wr_ref[...], preferred_element_type=_f32)


def _route_kern(lg_ref, gi1_ref, gi2_ref, w1_ref, w2_ref):
    lg = lg_ref[...]
    m = jnp.max(lg, axis=-1, keepdims=True)
    ex = jnp.exp(lg - m)
    p = ex / jnp.sum(ex, axis=-1, keepdims=True)
    lane = jax.lax.broadcasted_iota(jnp.int32, (S, E), 1)
    v1 = jnp.max(p, axis=-1, keepdims=True)
    i1 = jnp.min(jnp.where(p == v1, lane, E), axis=-1, keepdims=True)
    p2 = jnp.where(lane == i1, _f32(-1.0), p)
    v2 = jnp.max(p2, axis=-1, keepdims=True)
    i2 = jnp.min(jnp.where(p2 == v2, lane, E), axis=-1, keepdims=True)
    wsum = v1 + v2
    oh = (lane == i1).astype(_f32) + (lane == i2).astype(_f32)
    tri = (jax.lax.broadcasted_iota(jnp.int32, (S, S), 0)
           > jax.lax.broadcasted_iota(jnp.int32, (S, S), 1)).astype(_f32)
    cex = jnp.dot(tri, oh, preferred_element_type=_f32)
    pos1 = jnp.sum(jnp.where(lane == i1, cex, 0.0), axis=-1, keepdims=True).astype(jnp.int32)
    pos2 = jnp.sum(jnp.where(lane == i2, cex, 0.0), axis=-1, keepdims=True).astype(jnp.int32)
    keep1 = pos1 < C
    keep2 = pos2 < C
    gi1_ref[...] = jnp.where(keep1, i1 * C + pos1, EC)
    gi2_ref[...] = jnp.where(keep2, i2 * C + pos2, EC)
    w1_ref[...] = jnp.where(keep1, v1 / wsum, 0.0)
    w2_ref[...] = jnp.where(keep2, v2 / wsum, 0.0)


def _dispatch_kern(g1r_ref, g2r_ref, x_ref, buf_ref):
    sb = pl.program_id(0)
    srow = jax.lax.broadcasted_iota(jnp.int32, (SB, S), 0) + sb * SB
    P = ((g1r_ref[...] == srow).astype(_f32)
         + (g2r_ref[...] == srow).astype(_f32))
    buf_ref[...] = jnp.dot(P, x_ref[...], preferred_element_type=_f32)


def _ffn_kern(buf_ref, wg_ref, wu_ref, wd_ref, o_ref):
    f = pl.program_id(1)
    b = buf_ref[...]
    a = jnp.dot(b, wg_ref[0], preferred_element_type=_f32)
    u = jnp.dot(b, wu_ref[0], preferred_element_type=_f32)
    g = a / (1.0 + jnp.exp(-a)) * u
    contrib = jnp.dot(g, wd_ref[0], preferred_element_type=_f32)

    @pl.when(f == 0)
    def _():
        o_ref[...] = contrib

    @pl.when(f > 0)
    def _():
        o_ref[...] += contrib


def _combine_kern(oe_ref, g1_ref, g2_ref, w1_ref, w2_ref, h_ref, o_ref):
    scol = jax.lax.broadcasted_iota(jnp.int32, (BSQ, EC), 1)
    W = (jnp.where(g1_ref[...] == scol, w1_ref[...], 0.0)
         + jnp.where(g2_ref[...] == scol, w2_ref[...], 0.0))
    o_ref[...] = h_ref[...] + jnp.dot(W, oe_ref[...], preferred_element_type=_f32)


def kernel(hidden_states, attention_mask, position_ids, gamma1, gamma2,
           W_qkv, W_o, W_router, W_gate, W_up, W_down):
    del attention_mask
    x = hidden_states.reshape(S, D)
    pos = position_ids.reshape(S, 1)
    g1 = gamma1.reshape(1, D)
    g2 = gamma2.reshape(1, D)

    nq = S // BSQ
    q, kt, v3 = pl.pallas_call(
        _prologue_kern,
        grid=(nq,),
        in_specs=[
            pl.BlockSpec((BSQ, D), lambda i: (i, 0)),
            pl.BlockSpec((BSQ, 1), lambda i: (i, 0)),
            pl.BlockSpec((1, D), lambda i: (0, 0)),
            pl.BlockSpec((D, QKVW), lambda i: (0, 0)),
        ],
        out_specs=[
            pl.BlockSpec((BSQ, D), lambda i: (i, 0)),
            pl.BlockSpec((KV * HD, BSQ), lambda i: (0, i)),
            pl.BlockSpec((KV, BSQ, HD), lambda i: (0, i, 0)),
        ],
        out_shape=[
            jax.ShapeDtypeStruct((S, D), _f32),
            jax.ShapeDtypeStruct((KV * HD, S), _f32),
            jax.ShapeDtypeStruct((KV, S, HD), _f32),
        ],
    )(x, pos, g1, W_qkv)

    kt3 = kt.reshape(KV, HD, S)
    attn = pl.pallas_call(
        _attn_kern,
        grid=(HP, NQB, NJ),
        in_specs=[
            pl.BlockSpec((BQ, 2 * HD), lambda hp, i, j: (i, hp)),
            pl.BlockSpec((1, HD, BK), lambda hp, i, j: (hp // 2, 0, j)),
            pl.BlockSpec((1, BK, HD), lambda hp, i, j: (hp // 2, j, 0)),
        ],
        out_specs=pl.BlockSpec((BQ, 2 * HD), lambda hp, i, j: (i, hp)),
        out_shape=jax.ShapeDtypeStruct((S, D), _f32),
        scratch_shapes=[
            pltpu.VMEM((BQ, 2 * HD), _f32),
            pltpu.VMEM((BQ, 1), _f32),
            pltpu.VMEM((BQ, 1), _f32),
            pltpu.VMEM((BQ, 1), _f32),
            pltpu.VMEM((BQ, 1), _f32),
        ],
    )(q, kt3, v3)

    h, x2, logits = pl.pallas_call(
        _proj_kern,
        grid=(nq,),
        in_specs=[
            pl.BlockSpec((BSQ, D), lambda i: (i, 0)),
            pl.BlockSpec((D, D), lambda i: (0, 0)),
            pl.BlockSpec((BSQ, D), lambda i: (i, 0)),
            pl.BlockSpec((1, D), lambda i: (0, 0)),
            pl.BlockSpec((D, E), lambda i: (0, 0)),
        ],
        out_specs=[
            pl.BlockSpec((BSQ, D), lambda i: (i, 0)),
            pl.BlockSpec((BSQ, D), lambda i: (i, 0)),
            pl.BlockSpec((BSQ, E), lambda i: (i, 0)),
        ],
        out_shape=[
            jax.ShapeDtypeStruct((S, D), _f32),
            jax.ShapeDtypeStruct((S, D), _f32),
            jax.ShapeDtypeStruct((S, E), _f32),
        ],
    )(attn, W_o, x, g2, W_router)

    gi1, gi2, w1, w2 = pl.pallas_call(
        _route_kern,
        grid=(1,),
        in_specs=[pl.BlockSpec((S, E), lambda i: (0, 0))],
        out_specs=[
            pl.BlockSpec((S, 1), lambda i: (0, 0)),
            pl.BlockSpec((S, 1), lambda i: (0, 0)),
            pl.BlockSpec((S, 1), lambda i: (0, 0)),
            pl.BlockSpec((S, 1), lambda i: (0, 0)),
        ],
        out_shape=[
            jax.ShapeDtypeStruct((S, 1), jnp.int32),
            jax.ShapeDtypeStruct((S, 1), jnp.int32),
            jax.ShapeDtypeStruct((S, 1), _f32),
            jax.ShapeDtypeStruct((S, 1), _f32),
        ],
    )(logits)

    gi1r = gi1.reshape(1, S)
    gi2r = gi2.reshape(1, S)

    buf = pl.pallas_call(
        _dispatch_kern,
        grid=(EC // SB,),
        in_specs=[
            pl.BlockSpec((1, S), lambda i: (0, 0)),
            pl.BlockSpec((1, S), lambda i: (0, 0)),
            pl.BlockSpec((S, D), lambda i: (0, 0)),
        ],
        out_specs=pl.BlockSpec((SB, D), lambda i: (i, 0)),
        out_shape=jax.ShapeDtypeStruct((EC, D), _f32),
    )(gi1r, gi2r, x2)

    oe = pl.pallas_call(
        _ffn_kern,
        grid=(E, DFF // FB),
        in_specs=[
            pl.BlockSpec((C, D), lambda e, f: (e, 0)),
            pl.BlockSpec((1, D, FB), lambda e, f: (e, 0, f)),
            pl.BlockSpec((1, D, FB), lambda e, f: (e, 0, f)),
            pl.BlockSpec((1, FB, D), lambda e, f: (e, f, 0)),
        ],
        out_specs=pl.BlockSpec((C, D), lambda e, f: (e, 0)),
        out_shape=jax.ShapeDtypeStruct((EC, D), _f32),
    )(buf, W_gate, W_up, W_down)

    out = pl.pallas_call(
        _combine_kern,
        grid=(nq,),
        in_specs=[
            pl.BlockSpec((EC, D), lambda i: (0, 0)),
            pl.BlockSpec((BSQ, 1), lambda i: (i, 0)),
            pl.BlockSpec((BSQ, 1), lambda i: (i, 0)),
            pl.BlockSpec((BSQ, 1), lambda i: (i, 0)),
            pl.BlockSpec((BSQ, 1), lambda i: (i, 0)),
            pl.BlockSpec((BSQ, D), lambda i: (i, 0)),
        ],
        out_specs=pl.BlockSpec((BSQ, D), lambda i: (i, 0)),
        out_shape=jax.ShapeDtypeStruct((S, D), _f32),
    )(oe, gi1, gi2, w1, w2, h)

    return out.reshape(B, S, D)


# speedup vs baseline: 1.0698x; 1.0698x over previous
import functools

import jax
import jax.numpy as jnp
import numpy as np
from jax.experimental import pallas as pl
from jax.experimental.pallas import tpu as pltpu

B, S, D = 1, 2048, 1024
H, KV, HD = 16, 4, 64
E, K, DFF = 8, 2, 2048
C = 512
EC = E * C
CLIP = 8.0
ROPE = 500000.0
EPS = 1e-5
REP = H // KV
HALF = HD // 2

BSQ = 256
QKVW = D + 2 * KV * HD
FB = 512
SB = 512

_f32 = jnp.float32


def _roll_lanes(t, sh):
    return jnp.concatenate([t[:, sh:], t[:, :sh]], axis=1)


def _rope(t, cs, ss, nlanes, scale=1.0):
    within = jax.lax.broadcasted_iota(jnp.int32, (1, nlanes), 1) % HD
    reps = nlanes // HALF
    cosv = jnp.concatenate([cs] * reps, axis=1)
    sinv = jnp.concatenate([ss] * reps, axis=1)
    rot = jnp.where(within < HALF, -_roll_lanes(t, HALF), _roll_lanes(t, nlanes - HALF))
    return (t * cosv + rot * sinv) * _f32(scale)


def _prologue_kern(x_ref, pos_ref, g1_ref, wqkv_ref, q_ref, kt_ref, v3_ref):
    x = x_ref[...]
    mu = jnp.mean(x, axis=-1, keepdims=True)
    var = jnp.mean((x - mu) ** 2, axis=-1, keepdims=True)
    h = (x - mu) * jax.lax.rsqrt(var + EPS) * g1_ref[...]
    qkv = jnp.dot(h, wqkv_ref[...], preferred_element_type=_f32)
    qkv = jnp.clip(qkv, -CLIP, CLIP)
    pos_f = pos_ref[...].astype(_f32)
    j = jax.lax.broadcasted_iota(jnp.int32, (1, HALF), 1).astype(_f32)
    inv = jnp.exp(j * _f32(-np.log(ROPE) / HALF))
    theta = pos_f * inv
    cs = jnp.cos(theta)
    ss = jnp.sin(theta)
    q_ref[...] = _rope(qkv[:, :D], cs, ss, D, scale=1.0 / np.sqrt(HD))
    k = _rope(qkv[:, D:D + KV * HD], cs, ss, KV * HD)
    kt_ref[...] = k.T
    v = qkv[:, D + KV * HD:]
    v3_ref[...] = jnp.concatenate(
        [v[:, kh * HD:(kh + 1) * HD][None] for kh in range(KV)], axis=0)


BQ = 512
BK = 256
NQB = S // BQ
NJ = S // BK
GH = REP


def _attn_kern(q_ref, kt_ref, v_ref, o_ref, qp_scr, acc_scr, m_scr, l_scr):
    qb = pl.program_id(1)
    nj = (qb + 1) * (BQ // BK)
    qp_scr[...] = jnp.concatenate(
        [q_ref[:, h * HD:(h + 1) * HD] for h in range(GH)], axis=0)
    m_scr[...] = jnp.full((GH * BQ, 1), _f32(-1e30))
    l_scr[...] = jnp.zeros((GH * BQ, 1), _f32)
    acc_scr[...] = jnp.zeros((GH * BQ, HD), _f32)
    row = (jax.lax.broadcasted_iota(jnp.int32, (GH * BQ, BK), 0) & (BQ - 1)) + qb * BQ
    col = jax.lax.broadcasted_iota(jnp.int32, (GH * BQ, BK), 1)

    def body(j, _):
        ktj = kt_ref[0, j]
        vj = v_ref[0, j]
        s = jnp.dot(qp_scr[...], ktj, preferred_element_type=_f32)
        s = jnp.where(col + j * BK <= row, s, _f32(-1e9))
        m_old = m_scr[...]
        m_new = jnp.maximum(m_old, jnp.max(s, axis=-1, keepdims=True))
        corr = jnp.exp(m_old - m_new)
        p = jnp.exp(s - m_new)
        acc_scr[...] = acc_scr[...] * corr + jnp.dot(p, vj, preferred_element_type=_f32)
        l_scr[...] = l_scr[...] * corr + jnp.sum(p, axis=-1, keepdims=True)
        m_scr[...] = m_new
        return 0

    jax.lax.fori_loop(0, nj, body, 0)
    accn = (acc_scr[...] / l_scr[...]).reshape(GH, BQ, HD)
    o_ref[...] = jnp.concatenate([accn[i] for i in range(GH)], axis=1)


def _proj_kern(attn_ref, wo_ref, res_ref, g2_ref, wr_ref, h_ref, x2_ref, lg_ref):
    hh = res_ref[...] + jnp.dot(attn_ref[...], wo_ref[...], preferred_element_type=_f32)
    h_ref[...] = hh
    mu = jnp.mean(hh, axis=-1, keepdims=True)
    var = jnp.mean((hh - mu) ** 2, axis=-1, keepdims=True)
    x2 = (hh - mu) * jax.lax.rsqrt(var + EPS) * g2_ref[...]
    x2_ref[...] = x2
    lg_ref[...] = jnp.dot(x2, wr_ref[...], preferred_element_type=_f32)


def _route_kern(lg_ref, gi1_ref, gi2_ref, w1_ref, w2_ref):
    lg = lg_ref[...]
    m = jnp.max(lg, axis=-1, keepdims=True)
    ex = jnp.exp(lg - m)
    p = ex / jnp.sum(ex, axis=-1, keepdims=True)
    lane = jax.lax.broadcasted_iota(jnp.int32, (S, E), 1)
    v1 = jnp.max(p, axis=-1, keepdims=True)
    i1 = jnp.min(jnp.where(p == v1, lane, E), axis=-1, keepdims=True)
    p2 = jnp.where(lane == i1, _f32(-1.0), p)
    v2 = jnp.max(p2, axis=-1, keepdims=True)
    i2 = jnp.min(jnp.where(p2 == v2, lane, E), axis=-1, keepdims=True)
    wsum = v1 + v2
    oh = (lane == i1).astype(_f32) + (lane == i2).astype(_f32)
    tri = (jax.lax.broadcasted_iota(jnp.int32, (S, S), 0)
           > jax.lax.broadcasted_iota(jnp.int32, (S, S), 1)).astype(_f32)
    cex = jnp.dot(tri, oh, preferred_element_type=_f32)
    pos1 = jnp.sum(jnp.where(lane == i1, cex, 0.0), axis=-1, keepdims=True).astype(jnp.int32)
    pos2 = jnp.sum(jnp.where(lane == i2, cex, 0.0), axis=-1, keepdims=True).astype(jnp.int32)
    keep1 = pos1 < C
    keep2 = pos2 < C
    gi1_ref[...] = jnp.where(keep1, i1 * C + pos1, EC)
    gi2_ref[...] = jnp.where(keep2, i2 * C + pos2, EC)
    w1_ref[...] = jnp.where(keep1, v1 / wsum, 0.0)
    w2_ref[...] = jnp.where(keep2, v2 / wsum, 0.0)


def _dispatch_kern(g1r_ref, g2r_ref, x_ref, buf_ref):
    sb = pl.program_id(0)
    srow = jax.lax.broadcasted_iota(jnp.int32, (SB, S), 0) + sb * SB
    P = ((g1r_ref[...] == srow).astype(_f32)
         + (g2r_ref[...] == srow).astype(_f32))
    buf_ref[...] = jnp.dot(P, x_ref[...], preferred_element_type=_f32)


def _ffn_kern(buf_ref, wg_ref, wu_ref, wd_ref, o_ref):
    f = pl.program_id(1)
    b = buf_ref[...]
    a = jnp.dot(b, wg_ref[0], preferred_element_type=_f32)
    u = jnp.dot(b, wu_ref[0], preferred_element_type=_f32)
    g = a / (1.0 + jnp.exp(-a)) * u
    contrib = jnp.dot(g, wd_ref[0], preferred_element_type=_f32)

    @pl.when(f == 0)
    def _():
        o_ref[...] = contrib

    @pl.when(f > 0)
    def _():
        o_ref[...] += contrib


def _combine_kern(oe_ref, g1_ref, g2_ref, w1_ref, w2_ref, h_ref, o_ref):
    scol = jax.lax.broadcasted_iota(jnp.int32, (BSQ, EC), 1)
    W = (jnp.where(g1_ref[...] == scol, w1_ref[...], 0.0)
         + jnp.where(g2_ref[...] == scol, w2_ref[...], 0.0))
    o_ref[...] = h_ref[...] + jnp.dot(W, oe_ref[...], preferred_element_type=_f32)


def kernel(hidden_states, attention_mask, position_ids, gamma1, gamma2,
           W_qkv, W_o, W_router, W_gate, W_up, W_down):
    del attention_mask
    x = hidden_states.reshape(S, D)
    pos = position_ids.reshape(S, 1)
    g1 = gamma1.reshape(1, D)
    g2 = gamma2.reshape(1, D)

    nq = S // BSQ
    q, kt, v3 = pl.pallas_call(
        _prologue_kern,
        grid=(nq,),
        in_specs=[
            pl.BlockSpec((BSQ, D), lambda i: (i, 0)),
            pl.BlockSpec((BSQ, 1), lambda i: (i, 0)),
            pl.BlockSpec((1, D), lambda i: (0, 0)),
            pl.BlockSpec((D, QKVW), lambda i: (0, 0)),
        ],
        out_specs=[
            pl.BlockSpec((BSQ, D), lambda i: (i, 0)),
            pl.BlockSpec((KV * HD, BSQ), lambda i: (0, i)),
            pl.BlockSpec((KV, BSQ, HD), lambda i: (0, i, 0)),
        ],
        out_shape=[
            jax.ShapeDtypeStruct((S, D), _f32),
            jax.ShapeDtypeStruct((KV * HD, S), _f32),
            jax.ShapeDtypeStruct((KV, S, HD), _f32),
        ],
    )(x, pos, g1, W_qkv)

    kt4 = kt.reshape(KV, HD, NJ, BK).transpose(0, 2, 1, 3)
    v4 = v3.reshape(KV, NJ, BK, HD)
    attn = pl.pallas_call(
        _attn_kern,
        grid=(KV, NQB),
        in_specs=[
            pl.BlockSpec((BQ, GH * HD), lambda g, i: (i, g)),
            pl.BlockSpec((1, NJ, HD, BK), lambda g, i: (g, 0, 0, 0)),
            pl.BlockSpec((1, NJ, BK, HD), lambda g, i: (g, 0, 0, 0)),
        ],
        out_specs=pl.BlockSpec((BQ, GH * HD), lambda g, i: (i, g)),
        out_shape=jax.ShapeDtypeStruct((S, D), _f32),
        scratch_shapes=[
            pltpu.VMEM((GH * BQ, HD), _f32),
            pltpu.VMEM((GH * BQ, HD), _f32),
            pltpu.VMEM((GH * BQ, 1), _f32),
            pltpu.VMEM((GH * BQ, 1), _f32),
        ],
    )(q, kt4, v4)

    h, x2, logits = pl.pallas_call(
        _proj_kern,
        grid=(nq,),
        in_specs=[
            pl.BlockSpec((BSQ, D), lambda i: (i, 0)),
            pl.BlockSpec((D, D), lambda i: (0, 0)),
            pl.BlockSpec((BSQ, D), lambda i: (i, 0)),
            pl.BlockSpec((1, D), lambda i: (0, 0)),
            pl.BlockSpec((D, E), lambda i: (0, 0)),
        ],
        out_specs=[
            pl.BlockSpec((BSQ, D), lambda i: (i, 0)),
            pl.BlockSpec((BSQ, D), lambda i: (i, 0)),
            pl.BlockSpec((BSQ, E), lambda i: (i, 0)),
        ],
        out_shape=[
            jax.ShapeDtypeStruct((S, D), _f32),
            jax.ShapeDtypeStruct((S, D), _f32),
            jax.ShapeDtypeStruct((S, E), _f32),
        ],
    )(attn, W_o, x, g2, W_router)

    gi1, gi2, w1, w2 = pl.pallas_call(
        _route_kern,
        grid=(1,),
        in_specs=[pl.BlockSpec((S, E), lambda i: (0, 0))],
        out_specs=[
            pl.BlockSpec((S, 1), lambda i: (0, 0)),
            pl.BlockSpec((S, 1), lambda i: (0, 0)),
            pl.BlockSpec((S, 1), lambda i: (0, 0)),
            pl.BlockSpec((S, 1), lambda i: (0, 0)),
        ],
        out_shape=[
            jax.ShapeDtypeStruct((S, 1), jnp.int32),
            jax.ShapeDtypeStruct((S, 1), jnp.int32),
            jax.ShapeDtypeStruct((S, 1), _f32),
            jax.ShapeDtypeStruct((S, 1), _f32),
        ],
    )(logits)

    gi1r = gi1.reshape(1, S)
    gi2r = gi2.reshape(1, S)

    buf = pl.pallas_call(
        _dispatch_kern,
        grid=(EC // SB,),
        in_specs=[
            pl.BlockSpec((1, S), lambda i: (0, 0)),
            pl.BlockSpec((1, S), lambda i: (0, 0)),
            pl.BlockSpec((S, D), lambda i: (0, 0)),
        ],
        out_specs=pl.BlockSpec((SB, D), lambda i: (i, 0)),
        out_shape=jax.ShapeDtypeStruct((EC, D), _f32),
    )(gi1r, gi2r, x2)

    oe = pl.pallas_call(
        _ffn_kern,
        grid=(E, DFF // FB),
        in_specs=[
            pl.BlockSpec((C, D), lambda e, f: (e, 0)),
            pl.BlockSpec((1, D, FB), lambda e, f: (e, 0, f)),
            pl.BlockSpec((1, D, FB), lambda e, f: (e, 0, f)),
            pl.BlockSpec((1, FB, D), lambda e, f: (e, f, 0)),
        ],
        out_specs=pl.BlockSpec((C, D), lambda e, f: (e, 0)),
        out_shape=jax.ShapeDtypeStruct((EC, D), _f32),
    )(buf, W_gate, W_up, W_down)

    out = pl.pallas_call(
        _combine_kern,
        grid=(nq,),
        in_specs=[
            pl.BlockSpec((EC, D), lambda i: (0, 0)),
            pl.BlockSpec((BSQ, 1), lambda i: (i, 0)),
            pl.BlockSpec((BSQ, 1), lambda i: (i, 0)),
            pl.BlockSpec((BSQ, 1), lambda i: (i, 0)),
            pl.BlockSpec((BSQ, 1), lambda i: (i, 0)),
            pl.BlockSpec((BSQ, D), lambda i: (i, 0)),
        ],
        out_specs=pl.BlockSpec((BSQ, D), lambda i: (i, 0)),
        out_shape=jax.ShapeDtypeStruct((S, D), _f32),
    )(oe, gi1, gi2, w1, w2, h)

    return out.reshape(B, S, D)


# speedup vs baseline: 1.6035x; 1.4989x over previous
import functools

import jax
import jax.numpy as jnp
import numpy as np
from jax.experimental import pallas as pl
from jax.experimental.pallas import tpu as pltpu

B, S, D = 1, 2048, 1024
H, KV, HD = 16, 4, 64
E, K, DFF = 8, 2, 2048
C = 512
EC = E * C
CLIP = 8.0
ROPE = 500000.0
EPS = 1e-5
REP = H // KV
HALF = HD // 2

BSQ = 256
QKVW = D + 2 * KV * HD
FB = 512
SB = 512

_f32 = jnp.float32


def _roll_lanes(t, sh):
    return jnp.concatenate([t[:, sh:], t[:, :sh]], axis=1)


def _rope(t, cs, ss, nlanes, scale=1.0):
    within = jax.lax.broadcasted_iota(jnp.int32, (1, nlanes), 1) % HD
    reps = nlanes // HALF
    cosv = jnp.concatenate([cs] * reps, axis=1)
    sinv = jnp.concatenate([ss] * reps, axis=1)
    rot = jnp.where(within < HALF, -_roll_lanes(t, HALF), _roll_lanes(t, nlanes - HALF))
    return (t * cosv + rot * sinv) * _f32(scale)


def _prologue_kern(x_ref, pos_ref, g1_ref, wqkv_ref, q_ref, kt_ref, v3_ref):
    x = x_ref[...]
    mu = jnp.mean(x, axis=-1, keepdims=True)
    var = jnp.mean((x - mu) ** 2, axis=-1, keepdims=True)
    h = (x - mu) * jax.lax.rsqrt(var + EPS) * g1_ref[...]
    qkv = jnp.dot(h, wqkv_ref[...], preferred_element_type=_f32)
    qkv = jnp.clip(qkv, -CLIP, CLIP)
    pos_f = pos_ref[...].astype(_f32)
    j = jax.lax.broadcasted_iota(jnp.int32, (1, HALF), 1).astype(_f32)
    inv = jnp.exp(j * _f32(-np.log(ROPE) / HALF))
    theta = pos_f * inv
    cs = jnp.cos(theta)
    ss = jnp.sin(theta)
    q_ref[...] = _rope(qkv[:, :D], cs, ss, D, scale=1.0 / np.sqrt(HD))
    k = _rope(qkv[:, D:D + KV * HD], cs, ss, KV * HD)
    kt_ref[...] = k.T
    v = qkv[:, D + KV * HD:]
    v3_ref[...] = jnp.concatenate(
        [v[:, kh * HD:(kh + 1) * HD][None] for kh in range(KV)], axis=0)


BQ = 512
BK = 256
NQB = S // BQ
NJ = S // BK
GH = REP


def _attn_kern(q_ref, kt_ref, v_ref, o_ref, qp_scr, acc_scr, m_scr, l_scr):
    qb = pl.program_id(1)
    npair = qb + 1
    qp_scr[...] = jnp.concatenate(
        [q_ref[:, h * HD:(h + 1) * HD] for h in range(GH)], axis=0)
    row = (jax.lax.broadcasted_iota(jnp.int32, (GH * BQ, BK), 0) & (BQ - 1)) + qb * BQ
    col = jax.lax.broadcasted_iota(jnp.int32, (GH * BQ, BK), 1)
    qp = qp_scr[...]

    def body(t, _):
        ja = 2 * t
        jb = 2 * t + 1
        sa = jnp.dot(qp, kt_ref[0, ja], preferred_element_type=_f32)
        sb = jnp.dot(qp, kt_ref[0, jb], preferred_element_type=_f32)
        sa = jnp.where(col + ja * BK <= row, sa, _f32(-1e9))
        sb = jnp.where(col + jb * BK <= row, sb, _f32(-1e9))
        m_old = m_scr[...]
        m_new = jnp.maximum(
            m_old,
            jnp.maximum(jnp.max(sa, axis=-1, keepdims=True),
                        jnp.max(sb, axis=-1, keepdims=True)))
        pa = jnp.exp(sa - m_new)
        pb = jnp.exp(sb - m_new)
        pv = (jnp.dot(pa, v_ref[0, ja], preferred_element_type=_f32)
              + jnp.dot(pb, v_ref[0, jb], preferred_element_type=_f32))
        rs = (jnp.sum(pa, axis=-1, keepdims=True)
              + jnp.sum(pb, axis=-1, keepdims=True))
        corr = jnp.exp(m_old - m_new)

        @pl.when(t == 0)
        def _():
            acc_scr[...] = pv
            l_scr[...] = rs

        @pl.when(t > 0)
        def _():
            acc_scr[...] = acc_scr[...] * corr + pv
            l_scr[...] = l_scr[...] * corr + rs

        m_scr[...] = m_new
        return 0

    m_scr[...] = jnp.full((GH * BQ, 1), _f32(-1e30))
    jax.lax.fori_loop(0, npair, body, 0)
    accn = (acc_scr[...] / l_scr[...]).reshape(GH, BQ, HD)
    o_ref[...] = jnp.concatenate([accn[i] for i in range(GH)], axis=1)


def _proj_kern(attn_ref, wo_ref, res_ref, g2_ref, wr_ref, h_ref, x2_ref, lg_ref):
    hh = res_ref[...] + jnp.dot(attn_ref[...], wo_ref[...], preferred_element_type=_f32)
    h_ref[...] = hh
    mu = jnp.mean(hh, axis=-1, keepdims=True)
    var = jnp.mean((hh - mu) ** 2, axis=-1, keepdims=True)
    x2 = (hh - mu) * jax.lax.rsqrt(var + EPS) * g2_ref[...]
    x2_ref[...] = x2
    lg_ref[...] = jnp.dot(x2, wr_ref[...], preferred_element_type=_f32)


def _route_kern(lg_ref, gi1_ref, gi2_ref, w1_ref, w2_ref):
    lg = lg_ref[...]
    m = jnp.max(lg, axis=-1, keepdims=True)
    ex = jnp.exp(lg - m)
    p = ex / jnp.sum(ex, axis=-1, keepdims=True)
    lane = jax.lax.broadcasted_iota(jnp.int32, (S, E), 1)
    v1 = jnp.max(p, axis=-1, keepdims=True)
    i1 = jnp.min(jnp.where(p == v1, lane, E), axis=-1, keepdims=True)
    p2 = jnp.where(lane == i1, _f32(-1.0), p)
    v2 = jnp.max(p2, axis=-1, keepdims=True)
    i2 = jnp.min(jnp.where(p2 == v2, lane, E), axis=-1, keepdims=True)
    wsum = v1 + v2
    oh = (lane == i1).astype(_f32) + (lane == i2).astype(_f32)
    tri = (jax.lax.broadcasted_iota(jnp.int32, (S, S), 0)
           > jax.lax.broadcasted_iota(jnp.int32, (S, S), 1)).astype(_f32)
    cex = jnp.dot(tri, oh, preferred_element_type=_f32)
    pos1 = jnp.sum(jnp.where(lane == i1, cex, 0.0), axis=-1, keepdims=True).astype(jnp.int32)
    pos2 = jnp.sum(jnp.where(lane == i2, cex, 0.0), axis=-1, keepdims=True).astype(jnp.int32)
    keep1 = pos1 < C
    keep2 = pos2 < C
    gi1_ref[...] = jnp.where(keep1, i1 * C + pos1, EC)
    gi2_ref[...] = jnp.where(keep2, i2 * C + pos2, EC)
    w1_ref[...] = jnp.where(keep1, v1 / wsum, 0.0)
    w2_ref[...] = jnp.where(keep2, v2 / wsum, 0.0)


def _dispatch_kern(g1r_ref, g2r_ref, x_ref, buf_ref):
    sb = pl.program_id(0)
    srow = jax.lax.broadcasted_iota(jnp.int32, (SB, S), 0) + sb * SB
    P = ((g1r_ref[...] == srow).astype(_f32)
         + (g2r_ref[...] == srow).astype(_f32))
    buf_ref[...] = jnp.dot(P, x_ref[...], preferred_element_type=_f32)


def _ffn_kern(buf_ref, wg_ref, wu_ref, wd_ref, o_ref):
    f = pl.program_id(1)
    b = buf_ref[...]
    a = jnp.dot(b, wg_ref[0], preferred_element_type=_f32)
    u = jnp.dot(b, wu_ref[0], preferred_element_type=_f32)
    g = a / (1.0 + jnp.exp(-a)) * u
    contrib = jnp.dot(g, wd_ref[0], preferred_element_type=_f32)

    @pl.when(f == 0)
    def _():
        o_ref[...] = contrib

    @pl.when(f > 0)
    def _():
        o_ref[...] += contrib


def _combine_kern(oe_ref, g1_ref, g2_ref, w1_ref, w2_ref, h_ref, o_ref):
    scol = jax.lax.broadcasted_iota(jnp.int32, (BSQ, EC), 1)
    W = (jnp.where(g1_ref[...] == scol, w1_ref[...], 0.0)
         + jnp.where(g2_ref[...] == scol, w2_ref[...], 0.0))
    o_ref[...] = h_ref[...] + jnp.dot(W, oe_ref[...], preferred_element_type=_f32)


def kernel(hidden_states, attention_mask, position_ids, gamma1, gamma2,
           W_qkv, W_o, W_router, W_gate, W_up, W_down):
    del attention_mask
    x = hidden_states.reshape(S, D)
    pos = position_ids.reshape(S, 1)
    g1 = gamma1.reshape(1, D)
    g2 = gamma2.reshape(1, D)

    nq = S // BSQ
    q, kt, v3 = pl.pallas_call(
        _prologue_kern,
        grid=(nq,),
        in_specs=[
            pl.BlockSpec((BSQ, D), lambda i: (i, 0)),
            pl.BlockSpec((BSQ, 1), lambda i: (i, 0)),
            pl.BlockSpec((1, D), lambda i: (0, 0)),
            pl.BlockSpec((D, QKVW), lambda i: (0, 0)),
        ],
        out_specs=[
            pl.BlockSpec((BSQ, D), lambda i: (i, 0)),
            pl.BlockSpec((KV * HD, BSQ), lambda i: (0, i)),
            pl.BlockSpec((KV, BSQ, HD), lambda i: (0, i, 0)),
        ],
        out_shape=[
            jax.ShapeDtypeStruct((S, D), _f32),
            jax.ShapeDtypeStruct((KV * HD, S), _f32),
            jax.ShapeDtypeStruct((KV, S, HD), _f32),
        ],
    )(x, pos, g1, W_qkv)

    kt4 = kt.reshape(KV, HD, NJ, BK).transpose(0, 2, 1, 3)
    v4 = v3.reshape(KV, NJ, BK, HD)
    attn = pl.pallas_call(
        _attn_kern,
        grid=(KV, NQB),
        in_specs=[
            pl.BlockSpec((BQ, GH * HD), lambda g, i: (i, g)),
            pl.BlockSpec((1, NJ, HD, BK), lambda g, i: (g, 0, 0, 0)),
            pl.BlockSpec((1, NJ, BK, HD), lambda g, i: (g, 0, 0, 0)),
        ],
        out_specs=pl.BlockSpec((BQ, GH * HD), lambda g, i: (i, g)),
        out_shape=jax.ShapeDtypeStruct((S, D), _f32),
        scratch_shapes=[
            pltpu.VMEM((GH * BQ, HD), _f32),
            pltpu.VMEM((GH * BQ, HD), _f32),
            pltpu.VMEM((GH * BQ, 1), _f32),
            pltpu.VMEM((GH * BQ, 1), _f32),
        ],
    )(q, kt4, v4)

    h, x2, logits = pl.pallas_call(
        _proj_kern,
        grid=(nq,),
        in_specs=[
            pl.BlockSpec((BSQ, D), lambda i: (i, 0)),
            pl.BlockSpec((D, D), lambda i: (0, 0)),
            pl.BlockSpec((BSQ, D), lambda i: (i, 0)),
            pl.BlockSpec((1, D), lambda i: (0, 0)),
            pl.BlockSpec((D, E), lambda i: (0, 0)),
        ],
        out_specs=[
            pl.BlockSpec((BSQ, D), lambda i: (i, 0)),
            pl.BlockSpec((BSQ, D), lambda i: (i, 0)),
            pl.BlockSpec((BSQ, E), lambda i: (i, 0)),
        ],
        out_shape=[
            jax.ShapeDtypeStruct((S, D), _f32),
            jax.ShapeDtypeStruct((S, D), _f32),
            jax.ShapeDtypeStruct((S, E), _f32),
        ],
    )(attn, W_o, x, g2, W_router)

    gi1, gi2, w1, w2 = pl.pallas_call(
        _route_kern,
        grid=(1,),
        in_specs=[pl.BlockSpec((S, E), lambda i: (0, 0))],
        out_specs=[
            pl.BlockSpec((S, 1), lambda i: (0, 0)),
            pl.BlockSpec((S, 1), lambda i: (0, 0)),
            pl.BlockSpec((S, 1), lambda i: (0, 0)),
            pl.BlockSpec((S, 1), lambda i: (0, 0)),
        ],
        out_shape=[
            jax.ShapeDtypeStruct((S, 1), jnp.int32),
            jax.ShapeDtypeStruct((S, 1), jnp.int32),
            jax.ShapeDtypeStruct((S, 1), _f32),
            jax.ShapeDtypeStruct((S, 1), _f32),
        ],
    )(logits)

    gi1r = gi1.reshape(1, S)
    gi2r = gi2.reshape(1, S)

    buf = pl.pallas_call(
        _dispatch_kern,
        grid=(EC // SB,),
        in_specs=[
            pl.BlockSpec((1, S), lambda i: (0, 0)),
            pl.BlockSpec((1, S), lambda i: (0, 0)),
            pl.BlockSpec((S, D), lambda i: (0, 0)),
        ],
        out_specs=pl.BlockSpec((SB, D), lambda i: (i, 0)),
        out_shape=jax.ShapeDtypeStruct((EC, D), _f32),
    )(gi1r, gi2r, x2)

    oe = pl.pallas_call(
        _ffn_kern,
        grid=(E, DFF // FB),
        in_specs=[
            pl.BlockSpec((C, D), lambda e, f: (e, 0)),
            pl.BlockSpec((1, D, FB), lambda e, f: (e, 0, f)),
            pl.BlockSpec((1, D, FB), lambda e, f: (e, 0, f)),
            pl.BlockSpec((1, FB, D), lambda e, f: (e, f, 0)),
        ],
        out_specs=pl.BlockSpec((C, D), lambda e, f: (e, 0)),
        out_shape=jax.ShapeDtypeStruct((EC, D), _f32),
    )(buf, W_gate, W_up, W_down)

    out = pl.pallas_call(
        _combine_kern,
        grid=(nq,),
        in_specs=[
            pl.BlockSpec((EC, D), lambda i: (0, 0)),
            pl.BlockSpec((BSQ, 1), lambda i: (i, 0)),
            pl.BlockSpec((BSQ, 1), lambda i: (i, 0)),
            pl.BlockSpec((BSQ, 1), lambda i: (i, 0)),
            pl.BlockSpec((BSQ, 1), lambda i: (i, 0)),
            pl.BlockSpec((BSQ, D), lambda i: (i, 0)),
        ],
        out_specs=pl.BlockSpec((BSQ, D), lambda i: (i, 0)),
        out_shape=jax.ShapeDtypeStruct((S, D), _f32),
    )(oe, gi1, gi2, w1, w2, h)

    return out.reshape(B, S, D)
